# baseline (speedup 1.0000x reference)
import functools

import jax
import jax.numpy as jnp
from jax import lax
from jax.experimental import pallas as pl
from jax.experimental.pallas import tpu as pltpu, tpu_sc as plsc


def kernel(state, policy):
    (B,) = state.shape
    V, D = policy.shape
    info = plsc.get_sparse_core_info()
    nw = info.num_cores * info.num_subcores
    n_vreg = B // 16
    nb = -(-V // 128)
    nb_full = V // 128
    tail_w = V - nb_full * 128
    bpw = 256
    own_shift = 15
    n_bk = 272

    pt = policy.T

    mesh = plsc.VectorSubcoreMesh(core_axis_name="c", subcore_axis_name="s")

    @functools.partial(
        pl.kernel,
        mesh=mesh,
        compiler_params=pltpu.CompilerParams(needs_layout_passes=False),
        out_type=jax.ShapeDtypeStruct((B + 32, 128), jnp.float32),
        scratch_types=[
            pltpu.VMEM((B,), jnp.int32),
            pltpu.VMEM((B + 16,), jnp.int32),
            pltpu.VMEM((B + 16,), jnp.int32),
            pltpu.VMEM((B + 16,), jnp.int32),
            pltpu.VMEM((n_bk,), jnp.int32),
            pltpu.VMEM((n_bk,), jnp.int32),
            pltpu.VMEM((n_bk,), jnp.int32),
            pltpu.VMEM((5, D, 128), jnp.float32),
            pltpu.VMEM((D, tail_w), jnp.float32),
            pltpu.VMEM((2, 32, 128), jnp.float32),
            pltpu.VMEM((2, 32), jnp.int32),
            pltpu.SemaphoreType.DMA,
            pltpu.SemaphoreType.DMA,
        ],
    )
    def gather_k(idx_hbm, table_hbm, out_hbm, alli, myi, myr, sidx,
                 hist, offs0, offs_run, ring, endb, obuf, orow, sem_r, sem_o):
        w = lax.axis_index("s") * info.num_cores + lax.axis_index("c")
        iota = lax.iota(jnp.int32, 16)
        b_lo = w << 8
        b_hi = jnp.maximum(b_lo, jnp.minimum(b_lo + bpw, nb_full))

        def fire(b):
            start = pl.multiple_of(b * 128, 128)
            return pltpu.async_copy(
                table_hbm.at[:, pl.ds(start, 128)],
                ring.at[lax.rem(b - b_lo, 5)], sem_r)

        for d in range(4):
            @pl.when(b_lo + d < b_hi)
            def _():
                fire(b_lo + d)

        pltpu.sync_copy(idx_hbm, alli)
        trash = jnp.full((16,), B, jnp.int32) + iota
        for sl in range(2):
            orow[sl, pl.ds(0, 16)] = trash
            orow[sl, pl.ds(16, 16)] = trash
        for hv in range(n_bk // 16):
            hist[pl.ds(hv * 16, 16)] = jnp.zeros((16,), jnp.int32)

        def splat16(x):
            return jnp.full((16,), 0, jnp.int32) + x

        def route(q, cnt):
            iv = alli[pl.ds(q * 16, 16)]
            m = (iv >> own_shift) == w
            mi = m.astype(jnp.int32)
            s = jnp.sum(mi)

            @pl.when(s > 0)
            def _():
                pre = jnp.cumsum(mi)
                pos = jnp.where(m, cnt + pre - 1, B)
                plsc.store_scatter(myi, [pos], iv)
                plsc.store_scatter(myr, [pos], iota + q * 16)

            return cnt + s

        my_n = lax.fori_loop(0, n_vreg, route, 0)

        def histk(k, _):
            i_s = jnp.max(plsc.load_gather(myi, [splat16(k)]))
            bl = (i_s >> 7) - (w << 8)
            h = plsc.load_gather(hist, [splat16(bl)])
            plsc.store_scatter(hist, [splat16(bl)], h + 1)
            return 0

        lax.fori_loop(0, my_n, histk, 0)

        def pref(hv, carry):
            hvv = hist[pl.ds(hv * 16, 16)]
            inc = jnp.cumsum(hvv)
            exc = inc - hvv + carry
            offs0[pl.ds(hv * 16, 16)] = exc
            offs_run[pl.ds(hv * 16, 16)] = exc
            return carry + jnp.max(inc)

        lax.fori_loop(0, n_bk // 16, pref, 0)

        def place(k, _):
            isp = plsc.load_gather(myi, [splat16(k)])
            rsp = plsc.load_gather(myr, [splat16(k)])
            bl = (jnp.max(isp) >> 7) - (w << 8)
            p = jnp.max(plsc.load_gather(offs_run, [splat16(bl)]))
            plsc.store_scatter(offs_run, [splat16(bl)], splat16(p + 1))
            plsc.store_scatter(sidx, [splat16(p)], (rsp << 7) | (isp & 127))
            return 0

        lax.fori_loop(0, my_n, place, 0)

        def emit_row(buf, k, cnt_o):
            sp = plsc.load_gather(sidx, [splat16(k)])
            rsp = sp >> 7
            lane = sp & 127
            o = cnt_o & 31
            fsl = (cnt_o >> 5) & 1
            for jg in range(D // 16):
                jv = iota + jg * 16
                vals = plsc.load_gather(buf, [jv, lane])
                plsc.store_scatter(obuf.at[fsl], [splat16(o), jv], vals)
            plsc.store_scatter(orow.at[fsl], [splat16(o)], rsp)

            @pl.when(o == 31)
            def _():
                @pl.when(cnt_o >= 63)
                def _():
                    pltpu.make_async_copy(
                        obuf.at[0], out_hbm.at[orow.at[0]], sem_o).wait()

                pltpu.async_copy(obuf.at[fsl], out_hbm.at[orow.at[fsl]], sem_o)

            return cnt_o + 1

        def swp(b, cnt_o):
            @pl.when(b + 4 < b_hi)
            def _():
                fire(b + 4)

            pltpu.make_async_copy(
                table_hbm.at[:, pl.ds(0, 128)], ring.at[0], sem_r
            ).wait()
            bl = b - b_lo
            o0 = jnp.max(plsc.load_gather(offs0, [splat16(bl)]))
            o1 = jnp.max(plsc.load_gather(offs0, [splat16(bl + 1)]))
            slot = lax.rem(b - b_lo, 5)

            def row(k, cnt_o):
                return emit_row(ring.at[slot], k, cnt_o)

            return lax.fori_loop(o0, o1, row, cnt_o)

        cnt_o = lax.fori_loop(b_lo, b_hi, swp, 0)

        pltpu.sync_copy(table_hbm.at[:, pl.ds(nb_full * 128, tail_w)], endb)
        ebl = jnp.minimum(jnp.maximum(nb_full - b_lo, 0), n_bk - 2)
        e0 = jnp.max(plsc.load_gather(offs0, [splat16(ebl)]))
        e1 = jnp.max(plsc.load_gather(offs0, [splat16(ebl + 1)]))
        def erow(k, cnt_o):
            return emit_row(endb, k, cnt_o)

        cnt_o = lax.fori_loop(e0, e1, erow, cnt_o)

        rem = cnt_o & 31

        @pl.when(rem != 0)
        def _():
            fsl = (cnt_o >> 5) & 1
            pltpu.async_copy(obuf.at[fsl], out_hbm.at[orow.at[fsl]], sem_o)

        n_out = jnp.minimum(cnt_o >> 5, 1) + (rem != 0).astype(jnp.int32)

        def drain(_, acc):
            pltpu.make_async_copy(
                obuf.at[0], out_hbm.at[orow.at[0]], sem_o).wait()
            return acc

        lax.fori_loop(0, n_out, drain, 0)

    out128 = gather_k(state.astype(jnp.int32), pt)
    return out128[:B, :D]


# speedup vs baseline: 1.0143x; 1.0143x over previous
import functools

import jax
import jax.numpy as jnp
from jax import lax
from jax.experimental import pallas as pl
from jax.experimental.pallas import tpu as pltpu, tpu_sc as plsc


def kernel(state, policy):
    (B,) = state.shape
    V, D = policy.shape
    info = plsc.get_sparse_core_info()
    nw = info.num_cores * info.num_subcores
    n_vreg = B // 16
    nb = -(-V // 128)
    nb_full = V // 128
    tail_w = V - nb_full * 128
    bpw = 256
    own_shift = 15
    n_bk = 272

    pt = policy.T

    mesh = plsc.VectorSubcoreMesh(core_axis_name="c", subcore_axis_name="s")

    @functools.partial(
        pl.kernel,
        mesh=mesh,
        compiler_params=pltpu.CompilerParams(needs_layout_passes=False),
        out_type=jax.ShapeDtypeStruct((B + 32, 128), jnp.float32),
        scratch_types=[
            pltpu.VMEM((B,), jnp.int32),
            pltpu.VMEM((B + 16,), jnp.int32),
            pltpu.VMEM((B + 16,), jnp.int32),
            pltpu.VMEM((B + 16,), jnp.int32),
            pltpu.VMEM((n_bk,), jnp.int32),
            pltpu.VMEM((n_bk,), jnp.int32),
            pltpu.VMEM((n_bk,), jnp.int32),
            pltpu.VMEM((6, D, 128), jnp.float32),
            pltpu.VMEM((D, tail_w), jnp.float32),
            pltpu.VMEM((2, 16, 128), jnp.float32),
            pltpu.VMEM((2, 16), jnp.int32),
            pltpu.SemaphoreType.DMA,
            pltpu.SemaphoreType.DMA,
        ],
    )
    def gather_k(idx_hbm, table_hbm, out_hbm, alli, myi, myr, sidx,
                 hist, offs0, offs_run, ring, endb, obuf, orow, sem_r, sem_o):
        w = lax.axis_index("s") * info.num_cores + lax.axis_index("c")
        iota = lax.iota(jnp.int32, 16)
        b_lo = w << 8
        b_hi = jnp.maximum(b_lo, jnp.minimum(b_lo + bpw, nb_full))

        def fire(b):
            start = pl.multiple_of(b * 128, 128)
            return pltpu.async_copy(
                table_hbm.at[:, pl.ds(start, 128)],
                ring.at[lax.rem(b - b_lo, 6)], sem_r)

        for d in range(5):
            @pl.when(b_lo + d < b_hi)
            def _():
                fire(b_lo + d)

        pltpu.sync_copy(idx_hbm, alli)
        trash = jnp.full((16,), B, jnp.int32) + iota
        for sl in range(2):
            orow[sl, pl.ds(0, 16)] = trash
        for hv in range(n_bk // 16):
            hist[pl.ds(hv * 16, 16)] = jnp.zeros((16,), jnp.int32)

        def splat16(x):
            return jnp.full((16,), 0, jnp.int32) + x

        def route(q, cnt):
            iv = alli[pl.ds(q * 16, 16)]
            m = (iv >> own_shift) == w
            mi = m.astype(jnp.int32)
            s = jnp.sum(mi)

            @pl.when(s > 0)
            def _():
                pre = jnp.cumsum(mi)
                pos = jnp.where(m, cnt + pre - 1, B)
                plsc.store_scatter(myi, [pos], iv)
                plsc.store_scatter(myr, [pos], iota + q * 16)

            return cnt + s

        my_n = lax.fori_loop(0, n_vreg, route, 0)

        def histk(k, _):
            i_s = jnp.max(plsc.load_gather(myi, [splat16(k)]))
            bl = (i_s >> 7) - (w << 8)
            h = plsc.load_gather(hist, [splat16(bl)])
            plsc.store_scatter(hist, [splat16(bl)], h + 1)
            return 0

        lax.fori_loop(0, my_n, histk, 0)

        def pref(hv, carry):
            hvv = hist[pl.ds(hv * 16, 16)]
            inc = jnp.cumsum(hvv)
            exc = inc - hvv + carry
            offs0[pl.ds(hv * 16, 16)] = exc
            offs_run[pl.ds(hv * 16, 16)] = exc
            return carry + jnp.max(inc)

        lax.fori_loop(0, n_bk // 16, pref, 0)

        def place(k, _):
            isp = plsc.load_gather(myi, [splat16(k)])
            rsp = plsc.load_gather(myr, [splat16(k)])
            bl = (jnp.max(isp) >> 7) - (w << 8)
            p = jnp.max(plsc.load_gather(offs_run, [splat16(bl)]))
            plsc.store_scatter(offs_run, [splat16(bl)], splat16(p + 1))
            plsc.store_scatter(sidx, [splat16(p)], (rsp << 7) | (isp & 127))
            return 0

        lax.fori_loop(0, my_n, place, 0)

        def emit_row(buf, k, cnt_o):
            sp = plsc.load_gather(sidx, [splat16(k)])
            rsp = sp >> 7
            lane = sp & 127
            o = cnt_o & 15
            fsl = (cnt_o >> 4) & 1
            for jg in range(D // 16):
                jv = iota + jg * 16
                vals = plsc.load_gather(buf, [jv, lane])
                plsc.store_scatter(obuf.at[fsl], [splat16(o), jv], vals)
            plsc.store_scatter(orow.at[fsl], [splat16(o)], rsp)

            @pl.when(o == 15)
            def _():
                @pl.when(cnt_o >= 31)
                def _():
                    pltpu.make_async_copy(
                        obuf.at[0], out_hbm.at[orow.at[0]], sem_o).wait()

                pltpu.async_copy(obuf.at[fsl], out_hbm.at[orow.at[fsl]], sem_o)

            return cnt_o + 1

        def swp(b, cnt_o):
            @pl.when(b + 5 < b_hi)
            def _():
                fire(b + 5)

            pltpu.make_async_copy(
                table_hbm.at[:, pl.ds(0, 128)], ring.at[0], sem_r
            ).wait()
            bl = b - b_lo
            o0 = jnp.max(plsc.load_gather(offs0, [splat16(bl)]))
            o1 = jnp.max(plsc.load_gather(offs0, [splat16(bl + 1)]))
            slot = lax.rem(b - b_lo, 6)

            def row(k, cnt_o):
                return emit_row(ring.at[slot], k, cnt_o)

            return lax.fori_loop(o0, o1, row, cnt_o)

        cnt_o = lax.fori_loop(b_lo, b_hi, swp, 0)

        pltpu.sync_copy(table_hbm.at[:, pl.ds(nb_full * 128, tail_w)], endb)
        ebl = jnp.minimum(jnp.maximum(nb_full - b_lo, 0), n_bk - 2)
        e0 = jnp.max(plsc.load_gather(offs0, [splat16(ebl)]))
        e1 = jnp.max(plsc.load_gather(offs0, [splat16(ebl + 1)]))
        def erow(k, cnt_o):
            return emit_row(endb, k, cnt_o)

        cnt_o = lax.fori_loop(e0, e1, erow, cnt_o)

        rem = cnt_o & 15

        @pl.when(rem != 0)
        def _():
            fsl = (cnt_o >> 4) & 1
            pltpu.async_copy(obuf.at[fsl], out_hbm.at[orow.at[fsl]], sem_o)

        n_out = jnp.minimum(cnt_o >> 4, 1) + (rem != 0).astype(jnp.int32)

        def drain(_, acc):
            pltpu.make_async_copy(
                obuf.at[0], out_hbm.at[orow.at[0]], sem_o).wait()
            return acc

        lax.fori_loop(0, n_out, drain, 0)

    out128 = gather_k(state.astype(jnp.int32), pt)
    return out128[:B, :D]


# speedup vs baseline: 1.0898x; 1.0744x over previous
import functools

import jax
import jax.numpy as jnp
from jax import lax
from jax.experimental import pallas as pl
from jax.experimental.pallas import tpu as pltpu, tpu_sc as plsc


def kernel(state, policy):
    (B,) = state.shape
    V, D = policy.shape
    info = plsc.get_sparse_core_info()
    nw = info.num_cores * info.num_subcores
    n_vreg = B // 16
    nb = -(-V // 128)
    nb_full = V // 128
    tail_w = V - nb_full * 128
    bpw = 256
    own_shift = 15
    n_bk = 272

    pt = policy.T

    mesh = plsc.VectorSubcoreMesh(core_axis_name="c", subcore_axis_name="s")

    @functools.partial(
        pl.kernel,
        mesh=mesh,
        compiler_params=pltpu.CompilerParams(needs_layout_passes=False),
        out_type=jax.ShapeDtypeStruct((B + 32, 128), jnp.float32),
        scratch_types=[
            pltpu.VMEM((B,), jnp.int32),
            pltpu.VMEM((B + 16,), jnp.int32),
            pltpu.VMEM((B + 16,), jnp.int32),
            pltpu.VMEM((B + 16,), jnp.int32),
            pltpu.VMEM((n_bk,), jnp.int32),
            pltpu.VMEM((n_bk,), jnp.int32),
            pltpu.VMEM((n_bk,), jnp.int32),
            pltpu.VMEM((6, D, 128), jnp.float32),
            pltpu.VMEM((D, tail_w), jnp.float32),
            pltpu.VMEM((32, 128), jnp.float32),
            pltpu.VMEM((32,), jnp.int32),
            pltpu.SemaphoreType.DMA,
            pltpu.SemaphoreType.DMA,
        ],
    )
    def gather_k(idx_hbm, table_hbm, out_hbm, alli, myi, myr, sidx,
                 hist, offs0, offs_run, ring, endb, obuf, orow, sem_r, sem_o):
        w = lax.axis_index("s") * info.num_cores + lax.axis_index("c")
        iota = lax.iota(jnp.int32, 16)
        b_lo = w << 8
        b_hi = jnp.maximum(b_lo, jnp.minimum(b_lo + bpw, nb_full))

        def fire(b):
            start = pl.multiple_of(b * 128, 128)
            return pltpu.async_copy(
                table_hbm.at[:, pl.ds(start, 128)],
                ring.at[lax.rem(b - b_lo, 6)], sem_r)

        for d in range(5):
            @pl.when(b_lo + d < b_hi)
            def _():
                fire(b_lo + d)

        pltpu.sync_copy(idx_hbm, alli)
        trash = jnp.full((16,), B, jnp.int32) + iota
        orow[pl.ds(0, 16)] = trash
        orow[pl.ds(16, 16)] = trash
        for hv in range(n_bk // 16):
            hist[pl.ds(hv * 16, 16)] = jnp.zeros((16,), jnp.int32)

        def splat16(x):
            return jnp.full((16,), 0, jnp.int32) + x

        def route(q, cnt):
            iv = alli[pl.ds(q * 16, 16)]
            m = (iv >> own_shift) == w
            mi = m.astype(jnp.int32)
            pre = jnp.cumsum(mi)
            pos = jnp.where(m, cnt + pre - 1, B)
            plsc.store_scatter(myi, [pos], iv)
            plsc.store_scatter(myr, [pos], iota + q * 16)
            return cnt + jnp.sum(mi)

        my_n = lax.fori_loop(0, n_vreg, route, 0)

        def histk(k, _):
            i_s = jnp.max(plsc.load_gather(myi, [splat16(k)]))
            bl = (i_s >> 7) - (w << 8)
            h = plsc.load_gather(hist, [splat16(bl)])
            plsc.store_scatter(hist, [splat16(bl)], h + 1)
            return 0

        lax.fori_loop(0, my_n, histk, 0)

        def pref(hv, carry):
            hvv = hist[pl.ds(hv * 16, 16)]
            inc = jnp.cumsum(hvv)
            exc = inc - hvv + carry
            offs0[pl.ds(hv * 16, 16)] = exc
            offs_run[pl.ds(hv * 16, 16)] = exc
            return carry + jnp.max(inc)

        lax.fori_loop(0, n_bk // 16, pref, 0)

        def place(k, _):
            isp = plsc.load_gather(myi, [splat16(k)])
            rsp = plsc.load_gather(myr, [splat16(k)])
            bl = (jnp.max(isp) >> 7) - (w << 8)
            p = jnp.max(plsc.load_gather(offs_run, [splat16(bl)]))
            plsc.store_scatter(offs_run, [splat16(bl)], splat16(p + 1))
            plsc.store_scatter(sidx, [splat16(p)], (rsp << 7) | (isp & 127))
            return 0

        lax.fori_loop(0, my_n, place, 0)

        def emit_row(buf, k, cnt_o):
            sp = plsc.load_gather(sidx, [splat16(k)])
            rsp = sp >> 7
            lane = sp & 127
            o = cnt_o & 31
            for jg in range(D // 16):
                jv = iota + jg * 16
                vals = plsc.load_gather(buf, [jv, lane])
                plsc.store_scatter(obuf, [splat16(o), jv], vals)
            plsc.store_scatter(orow, [splat16(o)], rsp)

            @pl.when(o == 31)
            def _():
                pltpu.async_copy(obuf, out_hbm.at[orow], sem_o).wait()

            return cnt_o + 1

        def swp(b, cnt_o):
            @pl.when(b + 5 < b_hi)
            def _():
                fire(b + 5)

            pltpu.make_async_copy(
                table_hbm.at[:, pl.ds(0, 128)], ring.at[0], sem_r
            ).wait()
            bl = b - b_lo
            o0 = jnp.max(plsc.load_gather(offs0, [splat16(bl)]))
            o1 = jnp.max(plsc.load_gather(offs0, [splat16(bl + 1)]))
            slot = lax.rem(b - b_lo, 6)

            def row(k, cnt_o):
                return emit_row(ring.at[slot], k, cnt_o)

            return lax.fori_loop(o0, o1, row, cnt_o)

        cnt_o = lax.fori_loop(b_lo, b_hi, swp, 0)

        pltpu.sync_copy(table_hbm.at[:, pl.ds(nb_full * 128, tail_w)], endb)
        ebl = jnp.minimum(jnp.maximum(nb_full - b_lo, 0), n_bk - 2)
        e0 = jnp.max(plsc.load_gather(offs0, [splat16(ebl)]))
        e1 = jnp.max(plsc.load_gather(offs0, [splat16(ebl + 1)]))
        def erow(k, cnt_o):
            return emit_row(endb, k, cnt_o)

        cnt_o = lax.fori_loop(e0, e1, erow, cnt_o)

        @pl.when((cnt_o & 31) != 0)
        def _():
            pltpu.async_copy(obuf, out_hbm.at[orow], sem_o).wait()

    out128 = gather_k(state.astype(jnp.int32), pt)
    return out128[:B, :D]


# speedup vs baseline: 1.1322x; 1.0390x over previous
import functools

import jax
import jax.numpy as jnp
from jax import lax
from jax.experimental import pallas as pl
from jax.experimental.pallas import tpu as pltpu, tpu_sc as plsc


def kernel(state, policy):
    (B,) = state.shape
    V, D = policy.shape
    info = plsc.get_sparse_core_info()
    nw = info.num_cores * info.num_subcores
    n_vreg = B // 16
    nb = -(-V // 128)
    nb_full = V // 128
    tail_w = V - nb_full * 128
    bpw = 256
    own_shift = 15
    n_bk = 272

    pt = policy.T

    mesh = plsc.VectorSubcoreMesh(core_axis_name="c", subcore_axis_name="s")

    @functools.partial(
        pl.kernel,
        mesh=mesh,
        compiler_params=pltpu.CompilerParams(needs_layout_passes=False),
        out_type=jax.ShapeDtypeStruct((B + 32, 128), jnp.float32),
        scratch_types=[
            pltpu.VMEM((B,), jnp.int32),
            pltpu.VMEM((B + 16,), jnp.int32),
            pltpu.VMEM((B + 16,), jnp.int32),
            pltpu.VMEM((B + 16,), jnp.int32),
            pltpu.VMEM((n_bk,), jnp.int32),
            pltpu.VMEM((n_bk,), jnp.int32),
            pltpu.VMEM((n_bk,), jnp.int32),
            pltpu.VMEM((6, D, 128), jnp.float32),
            pltpu.VMEM((D, tail_w), jnp.float32),
            pltpu.VMEM((32, 128), jnp.float32),
            pltpu.VMEM((32,), jnp.int32),
            pltpu.SemaphoreType.DMA,
            pltpu.SemaphoreType.DMA,
        ],
    )
    def gather_k(idx_hbm, table_hbm, out_hbm, alli, myi, myr, sidx,
                 hist, offs0, offs_run, ring, endb, obuf, orow, sem_r, sem_o):
        w = lax.axis_index("s") * info.num_cores + lax.axis_index("c")
        iota = lax.iota(jnp.int32, 16)
        b_lo = w << 8
        b_hi = jnp.maximum(b_lo, jnp.minimum(b_lo + bpw, nb_full))

        def fire(b):
            start = pl.multiple_of(b * 128, 128)
            return pltpu.async_copy(
                table_hbm.at[:, pl.ds(start, 128)],
                ring.at[lax.rem(b - b_lo, 6)], sem_r)

        for d in range(5):
            @pl.when(b_lo + d < b_hi)
            def _():
                fire(b_lo + d)

        pltpu.sync_copy(idx_hbm, alli)
        trash = jnp.full((16,), B, jnp.int32) + iota
        orow[pl.ds(0, 16)] = trash
        orow[pl.ds(16, 16)] = trash
        for hv in range(n_bk // 16):
            hist[pl.ds(hv * 16, 16)] = jnp.zeros((16,), jnp.int32)

        def splat16(x):
            return jnp.full((16,), 0, jnp.int32) + x

        def route(q, cnt):
            iv = alli[pl.ds(q * 16, 16)]
            m = (iv >> own_shift) == w
            mi = m.astype(jnp.int32)
            pre = jnp.cumsum(mi)
            pos = jnp.where(m, cnt + pre - 1, B)
            plsc.store_scatter(myi, [pos], iv)
            plsc.store_scatter(myr, [pos], iota + q * 16)
            return cnt + jnp.sum(mi)

        my_n = lax.fori_loop(0, n_vreg, route, 0)

        def histk(k, _):
            i_s = jnp.max(plsc.load_gather(myi, [splat16(k)]))
            bl = (i_s >> 7) - (w << 8)
            h = plsc.load_gather(hist, [splat16(bl)])
            plsc.store_scatter(hist, [splat16(bl)], h + 1)
            return 0

        lax.fori_loop(0, my_n, histk, 0)

        def pref(hv, carry):
            hvv = hist[pl.ds(hv * 16, 16)]
            inc = jnp.cumsum(hvv)
            exc = inc - hvv + carry
            offs0[pl.ds(hv * 16, 16)] = exc
            offs_run[pl.ds(hv * 16, 16)] = exc
            return carry + jnp.max(inc)

        lax.fori_loop(0, n_bk // 16, pref, 0)

        def place(k, _):
            isp = plsc.load_gather(myi, [splat16(k)])
            rsp = plsc.load_gather(myr, [splat16(k)])
            bl = (jnp.max(isp) >> 7) - (w << 8)
            p = jnp.max(plsc.load_gather(offs_run, [splat16(bl)]))
            plsc.store_scatter(offs_run, [splat16(bl)], splat16(p + 1))
            plsc.store_scatter(sidx, [splat16(p)], (rsp << 7) | (isp & 127))
            return 0

        lax.fori_loop(0, my_n, place, 0)

        def emit_row(buf, k, cnt_o):
            sp = plsc.load_gather(sidx, [splat16(k)])
            rsp = sp >> 7
            lane = sp & 127
            o = cnt_o & 31
            for jg in range(D // 16):
                jv = iota + jg * 16
                vals = plsc.load_gather(buf, [jv, lane])
                plsc.store_scatter(obuf, [splat16(o), jv], vals)
            plsc.store_scatter(orow, [splat16(o)], rsp)

            @pl.when(o == 31)
            def _():
                pltpu.async_copy(obuf, out_hbm.at[orow], sem_o).wait()

            return cnt_o + 1

        def nonempty(bl):
            lo = jnp.max(plsc.load_gather(offs0, [splat16(bl)]))
            hi = jnp.max(plsc.load_gather(offs0, [splat16(bl + 1)]))
            return lo, hi

        def swp(b, cnt_o):
            bl = b - b_lo

            @pl.when(b + 5 < b_hi)
            def _():
                lo5, hi5 = nonempty(bl + 5)

                @pl.when(hi5 > lo5)
                def _():
                    fire(b + 5)

            o0, o1 = nonempty(bl)

            @pl.when((bl < 5) | (o1 > o0))
            def _():
                pltpu.make_async_copy(
                    table_hbm.at[:, pl.ds(0, 128)], ring.at[0], sem_r
                ).wait()

            slot = lax.rem(bl, 6)

            def row(k, cnt_o):
                return emit_row(ring.at[slot], k, cnt_o)

            return lax.fori_loop(o0, o1, row, cnt_o)

        cnt_o = lax.fori_loop(b_lo, b_hi, swp, 0)

        pltpu.sync_copy(table_hbm.at[:, pl.ds(nb_full * 128, tail_w)], endb)
        ebl = jnp.minimum(jnp.maximum(nb_full - b_lo, 0), n_bk - 2)
        e0 = jnp.max(plsc.load_gather(offs0, [splat16(ebl)]))
        e1 = jnp.max(plsc.load_gather(offs0, [splat16(ebl + 1)]))
        def erow(k, cnt_o):
            return emit_row(endb, k, cnt_o)

        cnt_o = lax.fori_loop(e0, e1, erow, cnt_o)

        @pl.when((cnt_o & 31) != 0)
        def _():
            pltpu.async_copy(obuf, out_hbm.at[orow], sem_o).wait()

    out128 = gather_k(state.astype(jnp.int32), pt)
    return out128[:B, :D]


# speedup vs baseline: 1.1375x; 1.0046x over previous
import functools

import jax
import jax.numpy as jnp
from jax import lax
from jax.experimental import pallas as pl
from jax.experimental.pallas import tpu as pltpu, tpu_sc as plsc


def kernel(state, policy):
    (B,) = state.shape
    V, D = policy.shape
    info = plsc.get_sparse_core_info()
    nw = info.num_cores * info.num_subcores
    n_vreg = B // 16
    nb = -(-V // 128)
    nb_full = V // 128
    tail_w = V - nb_full * 128
    n_bk = 272

    pt = policy.T

    mesh = plsc.VectorSubcoreMesh(core_axis_name="c", subcore_axis_name="s")

    @functools.partial(
        pl.kernel,
        mesh=mesh,
        compiler_params=pltpu.CompilerParams(needs_layout_passes=False),
        out_type=jax.ShapeDtypeStruct((B + 32, 128), jnp.float32),
        scratch_types=[
            pltpu.VMEM((B,), jnp.int32),
            pltpu.VMEM((B + 16,), jnp.int32),
            pltpu.VMEM((B + 16,), jnp.int32),
            pltpu.VMEM((B + 16,), jnp.int32),
            pltpu.VMEM((n_bk,), jnp.int32),
            pltpu.VMEM((n_bk,), jnp.int32),
            pltpu.VMEM((n_bk,), jnp.int32),
            pltpu.VMEM((6, D, 128), jnp.float32),
            pltpu.VMEM((D, tail_w), jnp.float32),
            pltpu.VMEM((32, 128), jnp.float32),
            pltpu.VMEM((32,), jnp.int32),
            pltpu.SemaphoreType.DMA,
            pltpu.SemaphoreType.DMA,
        ],
    )
    def gather_k(idx_hbm, table_hbm, out_hbm, alli, myi, myr, sidx,
                 hist, offs0, offs_run, ring, endb, obuf, orow, sem_r, sem_o):
        w = lax.axis_index("s") * info.num_cores + lax.axis_index("c")
        iota = lax.iota(jnp.int32, 16)
        khi = 244 + (w < 4).astype(jnp.int32)

        def fire(k):
            start = pl.multiple_of((k * 32 + w) * 128, 128)
            return pltpu.async_copy(
                table_hbm.at[:, pl.ds(start, 128)],
                ring.at[lax.rem(k, 6)], sem_r)

        for d in range(5):
            fire(d)

        pltpu.sync_copy(idx_hbm, alli)
        trash = jnp.full((16,), B, jnp.int32) + iota
        orow[pl.ds(0, 16)] = trash
        orow[pl.ds(16, 16)] = trash
        for hv in range(n_bk // 16):
            hist[pl.ds(hv * 16, 16)] = jnp.zeros((16,), jnp.int32)

        def splat16(x):
            return jnp.full((16,), 0, jnp.int32) + x

        def route(q, cnt):
            iv = alli[pl.ds(q * 16, 16)]
            m = ((iv >> 7) & 31) == w
            mi = m.astype(jnp.int32)
            pre = jnp.cumsum(mi)
            pos = jnp.where(m, cnt + pre - 1, B)
            plsc.store_scatter(myi, [pos], iv)
            plsc.store_scatter(myr, [pos], iota + q * 16)
            return cnt + jnp.sum(mi)

        my_n = lax.fori_loop(0, n_vreg, route, 0)

        def histk(k, _):
            i_s = jnp.max(plsc.load_gather(myi, [splat16(k)]))
            bl = i_s >> 12
            h = plsc.load_gather(hist, [splat16(bl)])
            plsc.store_scatter(hist, [splat16(bl)], h + 1)
            return 0

        lax.fori_loop(0, my_n, histk, 0)

        def pref(hv, carry):
            hvv = hist[pl.ds(hv * 16, 16)]
            inc = jnp.cumsum(hvv)
            exc = inc - hvv + carry
            offs0[pl.ds(hv * 16, 16)] = exc
            offs_run[pl.ds(hv * 16, 16)] = exc
            return carry + jnp.max(inc)

        lax.fori_loop(0, n_bk // 16, pref, 0)

        def place(k, _):
            isp = plsc.load_gather(myi, [splat16(k)])
            rsp = plsc.load_gather(myr, [splat16(k)])
            bl = jnp.max(isp) >> 12
            p = jnp.max(plsc.load_gather(offs_run, [splat16(bl)]))
            plsc.store_scatter(offs_run, [splat16(bl)], splat16(p + 1))
            plsc.store_scatter(sidx, [splat16(p)], (rsp << 7) | (isp & 127))
            return 0

        lax.fori_loop(0, my_n, place, 0)

        def emit_row(buf, k, cnt_o):
            sp = plsc.load_gather(sidx, [splat16(k)])
            rsp = sp >> 7
            lane = sp & 127
            o = cnt_o & 31
            for jg in range(D // 16):
                jv = iota + jg * 16
                vals = plsc.load_gather(buf, [jv, lane])
                plsc.store_scatter(obuf, [splat16(o), jv], vals)
            plsc.store_scatter(orow, [splat16(o)], rsp)

            @pl.when(o == 31)
            def _():
                pltpu.async_copy(obuf, out_hbm.at[orow], sem_o).wait()

            return cnt_o + 1

        def nonempty(bl):
            lo = jnp.max(plsc.load_gather(offs0, [splat16(bl)]))
            hi = jnp.max(plsc.load_gather(offs0, [splat16(bl + 1)]))
            return lo, hi

        def swp(k, cnt_o):
            @pl.when(k + 5 < khi)
            def _():
                lo5, hi5 = nonempty(k + 5)

                @pl.when(hi5 > lo5)
                def _():
                    fire(k + 5)

            o0, o1 = nonempty(k)

            @pl.when((k < 5) | (o1 > o0))
            def _():
                pltpu.make_async_copy(
                    table_hbm.at[:, pl.ds(0, 128)], ring.at[0], sem_r
                ).wait()

            slot = lax.rem(k, 6)

            def row(r, cnt_o):
                return emit_row(ring.at[slot], r, cnt_o)

            return lax.fori_loop(o0, o1, row, cnt_o)

        cnt_o = lax.fori_loop(0, khi, swp, 0)

        pltpu.sync_copy(table_hbm.at[:, pl.ds(nb_full * 128, tail_w)], endb)
        ebl = jnp.where(w == (nb_full & 31), nb_full >> 5, n_bk - 2)
        e0 = jnp.max(plsc.load_gather(offs0, [splat16(ebl)]))
        e1 = jnp.max(plsc.load_gather(offs0, [splat16(ebl + 1)]))
        def erow(k, cnt_o):
            return emit_row(endb, k, cnt_o)

        cnt_o = lax.fori_loop(e0, e1, erow, cnt_o)

        @pl.when((cnt_o & 31) != 0)
        def _():
            pltpu.async_copy(obuf, out_hbm.at[orow], sem_o).wait()

    out128 = gather_k(state.astype(jnp.int32), pt)
    return out128[:B, :D]
